# baseline (speedup 1.0000x reference)
import functools

import numpy as np
import jax
import jax.numpy as jnp
from jax import lax
from jax.experimental import pallas as pl
from jax.experimental.pallas import tpu as pltpu
from jax.experimental.pallas import tpu_sc as plsc


def _spiral_index_map(n_circle=3, n_sequence=8, steps=(1, 2, 3)):
    width = 2 * n_circle + 1
    ci = cj = n_circle
    circle_coords = {}
    for k in range(1, n_circle + 1):
        coords = []
        i, j = ci - k, cj
        coords.append((i, j))
        moves = ([(0, 1)] * k + [(1, 0)] * (2 * k) + [(0, -1)] * (2 * k)
                 + [(-1, 0)] * (2 * k) + [(0, 1)] * (k - 1))
        for di, dj in moves:
            i += di
            j += dj
            coords.append((i, j))
        for q, cd in enumerate(coords):
            circle_coords[(k, q)] = cd
    seq_len = 1 + sum(steps)
    idx = np.zeros((n_sequence, seq_len), dtype=np.int32)
    for c in range(n_sequence):
        idx[c, 0] = ci * width + cj
        off = 1
        for k in range(1, n_circle + 1):
            s = steps[k - 1]
            pos = list(range(s * c, s * c + s))
            if c % 2 == 1:
                pos = pos[::-1]
            for q in pos:
                i, j = circle_coords[(k, q)]
                idx[c, off] = i * width + j
                off += 1
    return idx.reshape(-1)


_IDX56 = _spiral_index_map()

_NC, _NS = 2, 16
_NW = _NC * _NS
_NB = 2
_VL = 16


@functools.cache
def _make_sc_scan(bs, c_int, w, n_band):
    n_seq = _IDX56.shape[0]
    assert c_int % _NB == 0
    cg_per_b = c_int // _NB
    assert (bs * cg_per_b) % _NW == 0
    n_groups = bs * cg_per_b // _NW
    assert n_groups % 2 == 0 and bs % _NW == 0
    b_per_w = bs // _NW

    offs = [k * _VL for k in range(n_band // _VL)]
    if n_band % _VL:
        offs.append(n_band - _VL)

    mesh = plsc.VectorSubcoreMesh(core_axis_name="c", subcore_axis_name="s",
                                  num_cores=_NC, num_subcores=_NS)

    @functools.partial(
        pl.kernel,
        out_type=jax.ShapeDtypeStruct((bs, c_int, 1, n_seq, n_band),
                                      jnp.float32),
        mesh=mesh,
        scratch_types=(
            [pltpu.VMEM((_NB, w, w, n_band), jnp.float32) for _ in range(2)]
            + [pltpu.VMEM((_NB, 1, n_seq, n_band), jnp.float32)
               for _ in range(2)]
            + [pltpu.SemaphoreType.DMA for _ in range(4)]
        ),
        compiler_params=pltpu.CompilerParams(use_tc_tiling_on_sc=True),
    )
    def sc_scan(x5, out, ib0, ib1, ob0, ob1, is0, is1, os0, os1):
        ibuf, obuf = (ib0, ib1), (ob0, ob1)
        isem, osem = (is0, is1), (os0, os1)
        wid = lax.axis_index("s") * _NC + lax.axis_index("c")
        b_base = wid * b_per_w

        def slots(g):
            return b_base + g // cg_per_b, (g % cg_per_b) * _NB

        def start_in(g, k):
            b, c0 = slots(g)
            pltpu.async_copy(x5.at[b, pl.ds(c0, _NB)], ibuf[k], isem[k])

        def wait_in(k):
            pltpu.make_async_copy(x5.at[b_base, pl.ds(0, _NB)], ibuf[k],
                                  isem[k]).wait()

        def start_out(g, k):
            b, c0 = slots(g)
            pltpu.async_copy(obuf[k], out.at[b, pl.ds(c0, _NB)], osem[k])

        def wait_out(k):
            pltpu.make_async_copy(obuf[k], out.at[b_base, pl.ds(0, _NB)],
                                  osem[k]).wait()

        start_in(0, 0)
        start_in(1, 1)

        @pl.loop(0, n_groups, step=2)
        def _grp(g0):
            for k in range(2):
                g = g0 + k
                wait_in(k)

                @pl.when(g >= 2)
                def _():
                    wait_out(k)

                for b in range(_NB):
                    for s in range(n_seq):
                        p = int(_IDX56[s])
                        for o in offs:
                            obuf[k][b, 0, s, pl.ds(o, _VL)] = (
                                ibuf[k][b, p // w, p % w, pl.ds(o, _VL)])
                start_out(g, k)

                @pl.when(g + 2 < n_groups)
                def _():
                    start_in(g + 2, k)

        wait_out(0)
        wait_out(1)

    return sc_scan


def kernel(x):
    bs, c_int, w, w2, n_band = x.shape
    return _make_sc_scan(bs, c_int, w, n_band)(x)
